# baseline (speedup 1.0000x reference)
import functools

import jax
import jax.numpy as jnp
from jax import lax
from jax.experimental import pallas as pl
from jax.experimental.pallas import tpu as pltpu
from jax.experimental.pallas import tpu_sc as plsc

M, K, N, E, TOPK = 2048, 1024, 1024, 8, 2
P = M * TOPK
BM = 256
NB = P // BM + E
PT = NB * BM

NC, NS = 2, 16
NW = NC * NS

RPW = PT // NW
ACH = 32
NACH = RPW // ACH

TPW = M // NW
TCH = 16
NTCH = TPW // TCH


def _sc_mesh():
    return plsc.VectorSubcoreMesh(
        core_axis_name="c", subcore_axis_name="s", num_cores=NC, num_subcores=NS
    )


def _wid():
    return lax.axis_index("s") * NC + lax.axis_index("c")


def _gather_body(src_hbm, a1_hbm, out_hbm, idx_v, buf0, buf1, sem0, sem1):
    base = _wid() * RPW
    pltpu.sync_copy(src_hbm.at[pl.ds(base, RPW)], idx_v)
    bufs = (buf0, buf1)
    sems = (sem0, sem1)
    copies = [None, None]
    copies[0] = pltpu.async_copy(a1_hbm.at[idx_v.at[pl.ds(0, ACH)]], buf0, sem0)
    for c in range(NACH):
        nxt = (c + 1) % 2
        if c + 1 < NACH:
            copies[nxt] = pltpu.async_copy(
                a1_hbm.at[idx_v.at[pl.ds((c + 1) * ACH, ACH)]], bufs[nxt], sems[nxt]
            )
        copies[c % 2].wait()
        pltpu.sync_copy(bufs[c % 2], out_hbm.at[pl.ds(base + c * ACH, ACH)])


def _sc_gather(src, a1):
    k = functools.partial(
        pl.kernel,
        out_type=jax.ShapeDtypeStruct((PT, K), jnp.float32),
        mesh=_sc_mesh(),
        scratch_types=[
            pltpu.VMEM((RPW,), jnp.int32),
            pltpu.VMEM((ACH, K), jnp.float32),
            pltpu.VMEM((ACH, K), jnp.float32),
            pltpu.SemaphoreType.DMA,
            pltpu.SemaphoreType.DMA,
        ],
    )(_gather_body)
    return k(src, a1)


def _combine_body(dest_hbm, tw0_hbm, tw1_hbm, osort_hbm, out_hbm,
                  idx_v, tw0_v, tw1_v, rbuf0, rbuf1, obuf, sem0, sem1):
    w = _wid()
    tbase = w * TPW
    pltpu.sync_copy(dest_hbm.at[pl.ds(tbase * TOPK, TPW * TOPK)], idx_v)
    pltpu.sync_copy(tw0_hbm.at[pl.ds(tbase, TPW)], tw0_v)
    pltpu.sync_copy(tw1_hbm.at[pl.ds(tbase, TPW)], tw1_v)
    bufs = (rbuf0, rbuf1)
    sems = (sem0, sem1)
    copies = [None, None]
    copies[0] = pltpu.async_copy(
        osort_hbm.at[idx_v.at[pl.ds(0, TCH * TOPK)]], rbuf0, sem0
    )
    for c in range(NTCH):
        nxt = (c + 1) % 2
        if c + 1 < NTCH:
            copies[nxt] = pltpu.async_copy(
                osort_hbm.at[idx_v.at[pl.ds((c + 1) * TCH * TOPK, TCH * TOPK)]],
                bufs[nxt], sems[nxt],
            )
        copies[c % 2].wait()
        rbuf = bufs[c % 2]
        w0c = tw0_v[pl.ds(c * TCH, 16)]
        w1c = tw1_v[pl.ds(c * TCH, 16)]
        for j in range(TCH):
            w0 = w0c[j]
            w1 = w1c[j]

            def lane_body(l, _, j=j, w0=w0, w1=w1):
                r0 = rbuf[2 * j, pl.ds(l * 16, 16)]
                r1 = rbuf[2 * j + 1, pl.ds(l * 16, 16)]
                obuf[j, pl.ds(l * 16, 16)] = w0 * r0 + w1 * r1
                return 0

            lax.fori_loop(0, K // 16, lane_body, 0, unroll=8)
        pltpu.sync_copy(obuf, out_hbm.at[pl.ds(tbase + c * TCH, TCH)])


def _sc_combine(dest, tw0, tw1, o_sorted):
    k = functools.partial(
        pl.kernel,
        out_type=jax.ShapeDtypeStruct((M, K), jnp.float32),
        mesh=_sc_mesh(),
        scratch_types=[
            pltpu.VMEM((TPW * TOPK,), jnp.int32),
            pltpu.VMEM((TPW,), jnp.float32),
            pltpu.VMEM((TPW,), jnp.float32),
            pltpu.VMEM((TCH * TOPK, K), jnp.float32),
            pltpu.VMEM((TCH * TOPK, K), jnp.float32),
            pltpu.VMEM((TCH, K), jnp.float32),
            pltpu.SemaphoreType.DMA,
            pltpu.SemaphoreType.DMA,
        ],
    )(_combine_body)
    return k(dest, tw0, tw1, o_sorted)


def _gemm_body(blk_e_ref, nb_ref, a_ref, w1_ref, w2_ref, o_ref):
    b = pl.program_id(0)

    @pl.when(b < nb_ref[0])
    def _():
        a = a_ref[...]
        h = lax.dot_general(a, w1_ref[0], (((1,), (1,)), ((), ())),
                            preferred_element_type=jnp.float32)
        gate = h[:, :N]
        up = h[:, N:]
        act = (gate * jax.nn.sigmoid(gate)) * up
        o_ref[...] = lax.dot_general(act, w2_ref[0], (((1,), (1,)), ((), ())),
                                     preferred_element_type=jnp.float32)


def _grouped_gemm(blk_e, nb_real, a_sorted, w1, w2):
    grid_spec = pltpu.PrefetchScalarGridSpec(
        num_scalar_prefetch=2,
        grid=(NB,),
        in_specs=[
            pl.BlockSpec((BM, K), lambda b, se, sn: (b, 0)),
            pl.BlockSpec((1, 2 * N, K), lambda b, se, sn: (se[b], 0, 0)),
            pl.BlockSpec((1, K, N), lambda b, se, sn: (se[b], 0, 0)),
        ],
        out_specs=pl.BlockSpec((BM, K), lambda b, se, sn: (b, 0)),
    )
    return pl.pallas_call(
        _gemm_body,
        grid_spec=grid_spec,
        out_shape=jax.ShapeDtypeStruct((PT, K), jnp.float32),
        compiler_params=pltpu.CompilerParams(
            dimension_semantics=("arbitrary",),
        ),
    )(blk_e, nb_real, a_sorted, w1, w2)


@jax.jit
def kernel(a1, w1, w2, topk_weights, topk_ids):
    ids = topk_ids.astype(jnp.int32)
    e_flat = ids.reshape(-1)
    onehot = (e_flat[:, None] == jnp.arange(E, dtype=jnp.int32)[None, :]
              ).astype(jnp.int32)
    incl = jnp.cumsum(onehot, axis=0)
    counts = incl[-1]
    rank = jnp.take_along_axis(incl - onehot, e_flat[:, None], axis=1)[:, 0]
    padded = ((counts + BM - 1) // BM) * BM
    ends = jnp.cumsum(padded)
    base = ends - padded
    dest = (base[e_flat] + rank).astype(jnp.int32)
    tok = jnp.arange(P, dtype=jnp.int32) // TOPK
    src = jnp.zeros((PT,), jnp.int32).at[dest].set(tok)
    blk_e = jnp.searchsorted(
        ends, jnp.arange(NB, dtype=jnp.int32) * BM, side="right"
    ).astype(jnp.int32)
    blk_e = jnp.minimum(blk_e, E - 1)
    nb_real = (ends[-1] // BM).astype(jnp.int32).reshape((1,))

    a_sorted = _sc_gather(src, a1)
    o_sorted = _grouped_gemm(blk_e, nb_real, a_sorted, w1, w2)
    tw0 = topk_weights[:, 0]
    tw1 = topk_weights[:, 1]
    out = _sc_combine(dest, tw0, tw1, o_sorted)
    return out
